# baseline (speedup 1.0000x reference)
import functools

import jax
import jax.numpy as jnp
from jax import lax
from jax.experimental import pallas as pl
from jax.experimental.pallas import tpu as pltpu
from jax.experimental.pallas import tpu_sc as plsc

N = 10000
NP = 10112
D_IN = 128
D_H = 64
E = 320000
NUM_CORES = 2
NUM_SUBCORES = 16
NW = NUM_CORES * NUM_SUBCORES
CHUNK = 112
EPW = E // NW
CPW = EPW // CHUNK
TAIL = EPW - CPW * CHUNK
ROWS_PER_SUB = NP // NUM_SUBCORES
NBUF = 4

_mesh = plsc.VectorSubcoreMesh(core_axis_name="c", subcore_axis_name="s")


@functools.partial(
    pl.kernel,
    out_type=(jax.ShapeDtypeStruct((NP, 16), jnp.float32),
              jax.ShapeDtypeStruct((NP, 16), jnp.float32)),
    mesh=_mesh,
    scratch_types=[
        pltpu.VMEM_SHARED((NP, 16), jnp.float32),
        pltpu.VMEM((EPW,), jnp.int32),
        pltpu.VMEM((CHUNK, 16), jnp.float32),
    ],
    compiler_params=pltpu.CompilerParams(use_tc_tiling_on_sc=False),
)
def _deg_sc(dst_hbm, zeros16_hbm, ones_hbm, out0_hbm, out1_hbm,
            deg_sh, didx, ones_v):
    c = lax.axis_index("c")
    s = lax.axis_index("s")
    wid = c * NUM_SUBCORES + s
    rsl = pl.ds(s * ROWS_PER_SUB, ROWS_PER_SUB)
    pltpu.sync_copy(zeros16_hbm.at[rsl], deg_sh.at[rsl])
    pltpu.sync_copy(ones_hbm, ones_v)
    pltpu.sync_copy(dst_hbm.at[pl.ds(wid * EPW, EPW)], didx)
    plsc.subcore_barrier()
    def body(j, _):
        pltpu.sync_copy(ones_v, deg_sh.at[didx.at[pl.ds(j * CHUNK, CHUNK)]],
                        add=True)
        return 0
    lax.fori_loop(0, CPW, body, 0)
    pltpu.sync_copy(ones_v.at[pl.ds(0, TAIL)],
                    deg_sh.at[didx.at[pl.ds(CPW * CHUNK, TAIL)]], add=True)
    plsc.subcore_barrier()

    @pl.when(c == 0)
    def _():
        pltpu.sync_copy(deg_sh.at[rsl], out0_hbm.at[rsl])

    @pl.when(c == 1)
    def _():
        pltpu.sync_copy(deg_sh.at[rsl], out1_hbm.at[rsl])


@functools.partial(
    pl.kernel,
    out_type=(jax.ShapeDtypeStruct((NP, D_H), jnp.float32),
              jax.ShapeDtypeStruct((NP, D_H), jnp.float32)),
    mesh=_mesh,
    scratch_types=[
        pltpu.VMEM_SHARED((NP, D_H), jnp.float32),
        pltpu.VMEM_SHARED((NP, D_H), jnp.float32),
        pltpu.VMEM((EPW,), jnp.int32),
        pltpu.VMEM((EPW,), jnp.int32),
        pltpu.VMEM((NBUF, CHUNK, D_H), jnp.float32),
        pltpu.SemaphoreType.DMA((NBUF,)),
        pltpu.SemaphoreType.DMA((NBUF,)),
    ],
    compiler_params=pltpu.CompilerParams(use_tc_tiling_on_sc=False),
)
def _scatter_sc(src_hbm, dst_hbm, hp_hbm, zeros64_hbm, out0_hbm, out1_hbm,
                acc_sh, hp_sh, sidx, didx, rows, gsem, ssem):
    c = lax.axis_index("c")
    s = lax.axis_index("s")
    wid = c * NUM_SUBCORES + s
    rsl = pl.ds(s * ROWS_PER_SUB, ROWS_PER_SUB)
    esl = pl.ds(wid * EPW, EPW)
    pltpu.async_copy(zeros64_hbm.at[rsl], acc_sh.at[rsl], gsem.at[0])
    pltpu.async_copy(hp_hbm.at[rsl], hp_sh.at[rsl], gsem.at[1])
    pltpu.async_copy(src_hbm.at[esl], sidx, gsem.at[2])
    pltpu.async_copy(dst_hbm.at[esl], didx, gsem.at[3])
    pltpu.make_async_copy(zeros64_hbm.at[rsl], acc_sh.at[rsl],
                          gsem.at[0]).wait()
    pltpu.make_async_copy(hp_hbm.at[rsl], hp_sh.at[rsl], gsem.at[1]).wait()
    pltpu.make_async_copy(src_hbm.at[esl], sidx, gsem.at[2]).wait()
    pltpu.make_async_copy(dst_hbm.at[esl], didx, gsem.at[3]).wait()
    plsc.subcore_barrier()

    def src_at(g):
        return sidx.at[pl.ds(g * CHUNK, CHUNK)]

    def dst_at(g):
        return didx.at[pl.ds(g * CHUNK, CHUNK)]

    for b in range(2):
        pltpu.async_copy(hp_sh.at[src_at(b)], rows.at[b], gsem.at[b])

    def body(g, _):
        b = g & (NBUF - 1)
        pltpu.make_async_copy(hp_sh.at[src_at(g)], rows.at[b],
                              gsem.at[b]).wait()
        pltpu.async_copy(rows.at[b], acc_sh.at[dst_at(g)], ssem.at[b],
                         add=True)

        @pl.when(g + 2 < CPW)
        def _():
            b2 = (g + 2) & (NBUF - 1)

            @pl.when(g >= 2)
            def _():
                pltpu.make_async_copy(rows.at[b2],
                                      acc_sh.at[dst_at(g - 2)],
                                      ssem.at[b2]).wait()

            pltpu.async_copy(hp_sh.at[src_at(g + 2)], rows.at[b2],
                             gsem.at[b2])
        return 0

    lax.fori_loop(0, CPW, body, 0)
    for g in range(CPW - 4, CPW):
        b = g % NBUF
        pltpu.make_async_copy(rows.at[b], acc_sh.at[dst_at(g)],
                              ssem.at[b]).wait()
    pltpu.sync_copy(hp_sh.at[sidx.at[pl.ds(CPW * CHUNK, TAIL)]],
                    rows.at[0, pl.ds(0, TAIL)])
    pltpu.sync_copy(rows.at[0, pl.ds(0, TAIL)],
                    acc_sh.at[didx.at[pl.ds(CPW * CHUNK, TAIL)]], add=True)
    plsc.subcore_barrier()

    @pl.when(c == 0)
    def _():
        pltpu.sync_copy(acc_sh.at[rsl], out0_hbm.at[rsl])

    @pl.when(c == 1)
    def _():
        pltpu.sync_copy(acc_sh.at[rsl], out1_hbm.at[rsl])


def _row_mask(shape):
    rows = lax.broadcasted_iota(jnp.int32, shape, 0)
    return rows < N


def _bn_masked(pre, g, b):
    mean = jnp.sum(pre, axis=0, keepdims=True) * (1.0 / N)
    var = jnp.sum(pre * pre, axis=0, keepdims=True) * (1.0 / N) - mean * mean
    return (pre - mean) * lax.rsqrt(var + 1e-5) * g + b


def _mm0_body(x_ref, w_ref, dp0_ref, dp1_ref, hp_ref, dinv_ref):
    h = jnp.dot(x_ref[...], w_ref[...], preferred_element_type=jnp.float32)
    deg = dp0_ref[:, 0:1] + dp1_ref[:, 0:1] + 1.0
    dinv = jnp.where(_row_mask((NP, 1)), lax.rsqrt(deg), 0.0)
    dinv_ref[...] = dinv
    hp_ref[...] = h * dinv


def _mm0(xp, W0, dp0, dp1):
    return pl.pallas_call(
        _mm0_body,
        out_shape=(jax.ShapeDtypeStruct((NP, D_H), jnp.float32),
                   jax.ShapeDtypeStruct((NP, 1), jnp.float32)),
    )(xp, W0, dp0, dp1)


def _combine_body(relu, has_res, acc0_ref, acc1_ref, hp_ref, dinv_ref, b_ref,
                  g_ref, be_ref, w_ref, res_ref, y_ref, hpn_ref):
    dinv = dinv_ref[...]
    pre = (acc0_ref[...] + acc1_ref[...] + hp_ref[...]) * dinv + b_ref[...]
    pre = jnp.where(_row_mask((NP, D_H)), pre, 0.0)
    y = _bn_masked(pre, g_ref[...], be_ref[...])
    if relu:
        y = jnp.maximum(y, 0.0)
    if has_res:
        y = y + res_ref[...]
    y_ref[...] = y
    hpn_ref[...] = jnp.dot(y, w_ref[...],
                           preferred_element_type=jnp.float32) * dinv


def _combine(acc, hp, dinv, b, g, be, Wn, res, relu):
    has_res = res is not None
    if res is None:
        res = hp
    return pl.pallas_call(
        functools.partial(_combine_body, relu, has_res),
        out_shape=(jax.ShapeDtypeStruct((NP, D_H), jnp.float32),
                   jax.ShapeDtypeStruct((NP, D_H), jnp.float32)),
    )(acc[0], acc[1], hp, dinv, b, g, be, Wn, res)


def _final_body(acc0_ref, acc1_ref, hp_ref, dinv_ref, b_ref, g_ref, be_ref,
                res_ref, w1_ref, b1_ref, g1_ref, be1_ref, w2_ref, b2_ref,
                g2_ref, be2_ref, w3_ref, b3_ref, z_ref):
    dinv = dinv_ref[...]
    pre = (acc0_ref[...] + acc1_ref[...] + hp_ref[...]) * dinv + b_ref[...]
    pre = jnp.where(_row_mask((NP, D_H)), pre, 0.0)
    h2 = _bn_masked(pre, g_ref[...], be_ref[...]) + res_ref[...]
    t1 = jnp.dot(h2, w1_ref[...], preferred_element_type=jnp.float32)
    t1 = jnp.where(_row_mask((NP, 4 * 16)), t1 + b1_ref[...], 0.0)
    z1 = jnp.maximum(_bn_masked(t1, g1_ref[...], be1_ref[...]), 0.0)
    t2 = jnp.dot(z1, w2_ref[...], preferred_element_type=jnp.float32)
    t2 = jnp.where(_row_mask((NP, 4 * 8)), t2 + b2_ref[...], 0.0)
    z2 = jnp.maximum(_bn_masked(t2, g2_ref[...], be2_ref[...]), 0.0)
    t3 = jnp.dot(z2, w3_ref[...], preferred_element_type=jnp.float32)
    t3 = t3 + b3_ref[...]
    z_ref[...] = 1.0 / (1.0 + jnp.exp(-t3))


def _final(acc, hp, dinv, b, g, be, res, w1, b1, g1, be1, w2, b2, g2, be2,
           w3, b3):
    return pl.pallas_call(
        _final_body,
        out_shape=jax.ShapeDtypeStruct((NP, 4), jnp.float32),
    )(acc[0], acc[1], hp, dinv, b, g, be, res, w1, b1, g1, be1, w2, b2, g2,
      be2, w3, b3)


def kernel(x, edge_index, batch_idx, node_type,
           W0, b0, g0, be0, W1, b1, g1, be1, W2, b2, g2, be2,
           cW1, cb1, cg1, cbe1, cW2, cb2, cg2, cbe2, cW3, cb3):
    del batch_idx, node_type
    f32 = jnp.float32
    xp = jnp.zeros((NP, D_IN), f32).at[:N].set(x)
    zeros16 = jnp.zeros((NP, 16), f32)
    zeros64 = jnp.zeros((NP, D_H), f32)

    b0r = b0.reshape(1, D_H)
    g0r = g0.reshape(1, D_H)
    be0r = be0.reshape(1, D_H)
    b1r = b1.reshape(1, D_H)
    g1r = g1.reshape(1, D_H)
    be1r = be1.reshape(1, D_H)
    b2r = b2.reshape(1, D_H)
    g2r = g2.reshape(1, D_H)
    be2r = be2.reshape(1, D_H)

    cw1f = cW1.transpose(1, 0, 2).reshape(D_H, 4 * 16)
    cb1f = cb1.reshape(1, 4 * 16)
    cg1f = cg1.reshape(1, 4 * 16)
    cbe1f = cbe1.reshape(1, 4 * 16)
    cw2bd = jax.scipy.linalg.block_diag(*[cW2[i] for i in range(4)])
    cb2f = cb2.reshape(1, 4 * 8)
    cg2f = cg2.reshape(1, 4 * 8)
    cbe2f = cbe2.reshape(1, 4 * 8)
    cw3bd = jax.scipy.linalg.block_diag(*[cW3[i] for i in range(4)])
    cb3f = cb3.reshape(1, 4)

    src = edge_index[0]
    dst = edge_index[1]
    ones16 = jnp.ones((CHUNK, 16), f32)
    deg_parts = _deg_sc(dst, zeros16, ones16)
    hp0, dinv = _mm0(xp, W0, *deg_parts)
    acc0 = _scatter_sc(src, dst, hp0, zeros64)
    h0, hp1 = _combine(acc0, hp0, dinv, b0r, g0r, be0r, W1, None, relu=True)
    acc1 = _scatter_sc(src, dst, hp1, zeros64)
    h1, hp2 = _combine(acc1, hp1, dinv, b1r, g1r, be1r, W2, h0, relu=True)
    acc2 = _scatter_sc(src, dst, hp2, zeros64)
    z = _final(acc2, hp2, dinv, b2r, g2r, be2r, h1,
               cw1f, cb1f, cg1f, cbe1f, cw2bd, cb2f, cg2f, cbe2f,
               cw3bd, cb3f)
    return z[:N]


# speedup vs baseline: 1.0292x; 1.0292x over previous
import functools

import jax
import jax.numpy as jnp
from jax import lax
from jax.experimental import pallas as pl
from jax.experimental.pallas import tpu as pltpu
from jax.experimental.pallas import tpu_sc as plsc

N = 10000
NP = 10112
D_IN = 128
D_H = 64
E = 320000
NUM_CORES = 2
NUM_SUBCORES = 16
NW = NUM_CORES * NUM_SUBCORES
CHUNK = 112
EPW = E // NW
CPW = EPW // CHUNK
TAIL = EPW - CPW * CHUNK
ROWS_PER_SUB = NP // NUM_SUBCORES
NBUF = 4

_mesh = plsc.VectorSubcoreMesh(core_axis_name="c", subcore_axis_name="s")


@functools.partial(
    pl.kernel,
    out_type=(jax.ShapeDtypeStruct((NP, 16), jnp.float32),
              jax.ShapeDtypeStruct((NP, 16), jnp.float32)),
    mesh=_mesh,
    scratch_types=[
        pltpu.VMEM_SHARED((NP, 16), jnp.float32),
        pltpu.VMEM((EPW,), jnp.int32),
        pltpu.VMEM((CHUNK, 16), jnp.float32),
    ],
    compiler_params=pltpu.CompilerParams(use_tc_tiling_on_sc=False),
)
def _deg_sc(edge_hbm, zeros16_hbm, ones_hbm, out0_hbm, out1_hbm,
            deg_sh, didx, ones_v):
    c = lax.axis_index("c")
    s = lax.axis_index("s")
    wid = c * NUM_SUBCORES + s
    rsl = pl.ds(s * ROWS_PER_SUB, ROWS_PER_SUB)
    pltpu.sync_copy(zeros16_hbm.at[rsl], deg_sh.at[rsl])
    pltpu.sync_copy(ones_hbm, ones_v)
    pltpu.sync_copy(edge_hbm.at[1, pl.ds(wid * EPW, EPW)], didx)
    plsc.subcore_barrier()
    def body(j, _):
        pltpu.sync_copy(ones_v, deg_sh.at[didx.at[pl.ds(j * CHUNK, CHUNK)]],
                        add=True)
        return 0
    lax.fori_loop(0, CPW, body, 0)
    pltpu.sync_copy(ones_v.at[pl.ds(0, TAIL)],
                    deg_sh.at[didx.at[pl.ds(CPW * CHUNK, TAIL)]], add=True)
    plsc.subcore_barrier()

    @pl.when(c == 0)
    def _():
        pltpu.sync_copy(deg_sh.at[rsl], out0_hbm.at[rsl])

    @pl.when(c == 1)
    def _():
        pltpu.sync_copy(deg_sh.at[rsl], out1_hbm.at[rsl])


@functools.partial(
    pl.kernel,
    out_type=(jax.ShapeDtypeStruct((NP, D_H), jnp.float32),
              jax.ShapeDtypeStruct((NP, D_H), jnp.float32)),
    mesh=_mesh,
    scratch_types=[
        pltpu.VMEM_SHARED((NP, D_H), jnp.float32),
        pltpu.VMEM_SHARED((NP, D_H), jnp.float32),
        pltpu.VMEM((EPW,), jnp.int32),
        pltpu.VMEM((EPW,), jnp.int32),
        pltpu.VMEM((NBUF, CHUNK, D_H), jnp.float32),
        pltpu.SemaphoreType.DMA((NBUF,)),
        pltpu.SemaphoreType.DMA((NBUF,)),
    ],
    compiler_params=pltpu.CompilerParams(use_tc_tiling_on_sc=False),
)
def _scatter_sc(edge_hbm, hp_hbm, zeros64_hbm, out0_hbm, out1_hbm,
                acc_sh, hp_sh, sidx, didx, rows, gsem, ssem):
    c = lax.axis_index("c")
    s = lax.axis_index("s")
    wid = c * NUM_SUBCORES + s
    rsl = pl.ds(s * ROWS_PER_SUB, ROWS_PER_SUB)
    esl = pl.ds(wid * EPW, EPW)
    pltpu.async_copy(zeros64_hbm.at[rsl], acc_sh.at[rsl], gsem.at[0])
    pltpu.async_copy(hp_hbm.at[rsl], hp_sh.at[rsl], gsem.at[1])
    pltpu.async_copy(edge_hbm.at[0, esl], sidx, gsem.at[2])
    pltpu.async_copy(edge_hbm.at[1, esl], didx, gsem.at[3])
    pltpu.make_async_copy(zeros64_hbm.at[rsl], acc_sh.at[rsl],
                          gsem.at[0]).wait()
    pltpu.make_async_copy(hp_hbm.at[rsl], hp_sh.at[rsl], gsem.at[1]).wait()
    pltpu.make_async_copy(edge_hbm.at[0, esl], sidx, gsem.at[2]).wait()
    pltpu.make_async_copy(edge_hbm.at[1, esl], didx, gsem.at[3]).wait()
    plsc.subcore_barrier()

    def src_at(g):
        return sidx.at[pl.ds(g * CHUNK, CHUNK)]

    def dst_at(g):
        return didx.at[pl.ds(g * CHUNK, CHUNK)]

    for b in range(2):
        pltpu.async_copy(hp_sh.at[src_at(b)], rows.at[b], gsem.at[b])

    def body(g, _):
        b = g & (NBUF - 1)
        pltpu.make_async_copy(hp_sh.at[src_at(g)], rows.at[b],
                              gsem.at[b]).wait()
        pltpu.async_copy(rows.at[b], acc_sh.at[dst_at(g)], ssem.at[b],
                         add=True)

        @pl.when(g + 2 < CPW)
        def _():
            b2 = (g + 2) & (NBUF - 1)

            @pl.when(g >= 2)
            def _():
                pltpu.make_async_copy(rows.at[b2],
                                      acc_sh.at[dst_at(g - 2)],
                                      ssem.at[b2]).wait()

            pltpu.async_copy(hp_sh.at[src_at(g + 2)], rows.at[b2],
                             gsem.at[b2])
        return 0

    lax.fori_loop(0, CPW, body, 0)
    for g in range(CPW - 4, CPW):
        b = g % NBUF
        pltpu.make_async_copy(rows.at[b], acc_sh.at[dst_at(g)],
                              ssem.at[b]).wait()
    pltpu.sync_copy(hp_sh.at[sidx.at[pl.ds(CPW * CHUNK, TAIL)]],
                    rows.at[0, pl.ds(0, TAIL)])
    pltpu.sync_copy(rows.at[0, pl.ds(0, TAIL)],
                    acc_sh.at[didx.at[pl.ds(CPW * CHUNK, TAIL)]], add=True)
    plsc.subcore_barrier()

    @pl.when(c == 0)
    def _():
        pltpu.sync_copy(acc_sh.at[rsl], out0_hbm.at[rsl])

    @pl.when(c == 1)
    def _():
        pltpu.sync_copy(acc_sh.at[rsl], out1_hbm.at[rsl])


def _row_mask(shape):
    rows = lax.broadcasted_iota(jnp.int32, shape, 0)
    return rows < N


def _bn_masked(pre, g, b):
    mean = jnp.sum(pre, axis=0, keepdims=True) * (1.0 / N)
    var = jnp.sum(pre * pre, axis=0, keepdims=True) * (1.0 / N) - mean * mean
    return (pre - mean) * lax.rsqrt(var + 1e-5) * g + b


def _mm0_body(x_ref, w_ref, dp0_ref, dp1_ref, hp_ref, dinv_ref):
    h = jnp.dot(x_ref[...], w_ref[...], preferred_element_type=jnp.float32)
    deg = dp0_ref[:, 0:1] + dp1_ref[:, 0:1] + 1.0
    dinv = jnp.where(_row_mask((NP, 1)), lax.rsqrt(deg), 0.0)
    dinv_ref[...] = dinv
    hp_ref[...] = h * dinv


def _mm0(xp, W0, dp0, dp1):
    return pl.pallas_call(
        _mm0_body,
        out_shape=(jax.ShapeDtypeStruct((NP, D_H), jnp.float32),
                   jax.ShapeDtypeStruct((NP, 1), jnp.float32)),
    )(xp, W0, dp0, dp1)


def _combine_body(relu, has_res, acc0_ref, acc1_ref, hp_ref, dinv_ref, b_ref,
                  g_ref, be_ref, w_ref, res_ref, y_ref, hpn_ref):
    dinv = dinv_ref[...]
    pre = (acc0_ref[...] + acc1_ref[...] + hp_ref[...]) * dinv + b_ref[...]
    pre = jnp.where(_row_mask((NP, D_H)), pre, 0.0)
    y = _bn_masked(pre, g_ref[...], be_ref[...])
    if relu:
        y = jnp.maximum(y, 0.0)
    if has_res:
        y = y + res_ref[...]
    y_ref[...] = y
    hpn_ref[...] = jnp.dot(y, w_ref[...],
                           preferred_element_type=jnp.float32) * dinv


def _combine(acc, hp, dinv, b, g, be, Wn, res, relu):
    has_res = res is not None
    if res is None:
        res = hp
    return pl.pallas_call(
        functools.partial(_combine_body, relu, has_res),
        out_shape=(jax.ShapeDtypeStruct((NP, D_H), jnp.float32),
                   jax.ShapeDtypeStruct((NP, D_H), jnp.float32)),
    )(acc[0], acc[1], hp, dinv, b, g, be, Wn, res)


def _final_body(acc0_ref, acc1_ref, hp_ref, dinv_ref, b_ref, g_ref, be_ref,
                res_ref, w1_ref, b1_ref, g1_ref, be1_ref, w2_ref, b2_ref,
                g2_ref, be2_ref, w3_ref, b3_ref, z_ref):
    dinv = dinv_ref[...]
    pre = (acc0_ref[...] + acc1_ref[...] + hp_ref[...]) * dinv + b_ref[...]
    pre = jnp.where(_row_mask((NP, D_H)), pre, 0.0)
    h2 = _bn_masked(pre, g_ref[...], be_ref[...]) + res_ref[...]
    t1 = jnp.dot(h2, w1_ref[...], preferred_element_type=jnp.float32)
    t1 = jnp.where(_row_mask((NP, 4 * 16)), t1 + b1_ref[...], 0.0)
    z1 = jnp.maximum(_bn_masked(t1, g1_ref[...], be1_ref[...]), 0.0)
    t2 = jnp.dot(z1, w2_ref[...], preferred_element_type=jnp.float32)
    t2 = jnp.where(_row_mask((NP, 4 * 8)), t2 + b2_ref[...], 0.0)
    z2 = jnp.maximum(_bn_masked(t2, g2_ref[...], be2_ref[...]), 0.0)
    t3 = jnp.dot(z2, w3_ref[...], preferred_element_type=jnp.float32)
    t3 = t3 + b3_ref[...]
    z_ref[...] = 1.0 / (1.0 + jnp.exp(-t3))


def _final(acc, hp, dinv, b, g, be, res, w1, b1, g1, be1, w2, b2, g2, be2,
           w3, b3):
    return pl.pallas_call(
        _final_body,
        out_shape=jax.ShapeDtypeStruct((NP, 4), jnp.float32),
    )(acc[0], acc[1], hp, dinv, b, g, be, res, w1, b1, g1, be1, w2, b2, g2,
      be2, w3, b3)


def kernel(x, edge_index, batch_idx, node_type,
           W0, b0, g0, be0, W1, b1, g1, be1, W2, b2, g2, be2,
           cW1, cb1, cg1, cbe1, cW2, cb2, cg2, cbe2, cW3, cb3):
    del batch_idx, node_type
    f32 = jnp.float32
    xp = jnp.zeros((NP, D_IN), f32).at[:N].set(x)
    zeros16 = jnp.zeros((NP, 16), f32)
    zeros64 = jnp.zeros((NP, D_H), f32)

    b0r = b0.reshape(1, D_H)
    g0r = g0.reshape(1, D_H)
    be0r = be0.reshape(1, D_H)
    b1r = b1.reshape(1, D_H)
    g1r = g1.reshape(1, D_H)
    be1r = be1.reshape(1, D_H)
    b2r = b2.reshape(1, D_H)
    g2r = g2.reshape(1, D_H)
    be2r = be2.reshape(1, D_H)

    cw1f = cW1.transpose(1, 0, 2).reshape(D_H, 4 * 16)
    cb1f = cb1.reshape(1, 4 * 16)
    cg1f = cg1.reshape(1, 4 * 16)
    cbe1f = cbe1.reshape(1, 4 * 16)
    cw2bd = jax.scipy.linalg.block_diag(*[cW2[i] for i in range(4)])
    cb2f = cb2.reshape(1, 4 * 8)
    cg2f = cg2.reshape(1, 4 * 8)
    cbe2f = cbe2.reshape(1, 4 * 8)
    cw3bd = jax.scipy.linalg.block_diag(*[cW3[i] for i in range(4)])
    cb3f = cb3.reshape(1, 4)

    ones16 = jnp.ones((CHUNK, 16), f32)
    deg_parts = _deg_sc(edge_index, zeros16, ones16)
    hp0, dinv = _mm0(xp, W0, *deg_parts)
    acc0 = _scatter_sc(edge_index, hp0, zeros64)
    h0, hp1 = _combine(acc0, hp0, dinv, b0r, g0r, be0r, W1, None, relu=True)
    acc1 = _scatter_sc(edge_index, hp1, zeros64)
    h1, hp2 = _combine(acc1, hp1, dinv, b1r, g1r, be1r, W2, h0, relu=True)
    acc2 = _scatter_sc(edge_index, hp2, zeros64)
    z = _final(acc2, hp2, dinv, b2r, g2r, be2r, h1,
               cw1f, cb1f, cg1f, cbe1f, cw2bd, cb2f, cg2f, cbe2f,
               cw3bd, cb3f)
    return z[:N]
